# Initial kernel scaffold; baseline (speedup 1.0000x reference)
import jax
import jax.numpy as jnp
from jax.experimental import pallas as pl


def kernel(conv_w_0, conv_b_0, conv_w_1, conv_b_1, conv_w_2, conv_b_2, conv_w_3, conv_b_3, conv_w_4, conv_b_4, conv_w_5, conv_b_5, conv_w_6, conv_b_6, conv_w_7, conv_b_7, conv_w_8, conv_b_8, conv_w_9, conv_b_9, conv_w_10, conv_b_10, conv_w_11, conv_b_11, conv_w_12, conv_b_12, fc_w_0, fc_b_0, fc_w_1, fc_b_1, fc_w_2, fc_b_2, x):
    raise NotImplementedError("write your pallas kernel here")


# speedup vs baseline: 1.3755x; 1.3755x over previous
import functools

import jax
import jax.numpy as jnp
from jax.experimental import pallas as pl
from jax.experimental.pallas import tpu as pltpu

_CFG = [64, 64, 'M', 128, 128, 'M', 256, 256, 256, 'M',
        512, 512, 512, 'M', 512, 512, 512, 'M']

_VMEM_LIMIT = 48 * 1024 * 1024


def _conv_layer(x, w, b):
    Bg, H, W, Cin = x.shape
    Cout = w.shape[3]
    Mp = Bg * (H + 2) * W
    xp = jnp.pad(x, ((0, 0), (1, 1), (1, 1), (0, 0)))
    ztail = jnp.zeros((2 * W, Cin), jnp.bfloat16)
    acc = jnp.zeros((Mp, Cout), jnp.float32)
    for kw in range(3):
        xs = xp[:, :, kw:kw + W, :].reshape(Mp, Cin)
        xs = jnp.concatenate([xs, ztail], axis=0)
        for kh in range(3):
            acc = acc + jnp.dot(xs[kh * W:kh * W + Mp, :], w[kw, kh],
                                preferred_element_type=jnp.float32)
    out = jnp.maximum(acc + b, 0.0).astype(jnp.bfloat16)
    return out.reshape(Bg, H + 2, W, Cout)[:, :H]


def _pool_layer(x):
    Bg, H, W, C = x.shape
    xr = x.reshape(Bg, H // 2, 2, W, C)
    v = jnp.maximum(xr[:, :, 0], xr[:, :, 1])
    vr = v.reshape(Bg, H // 2, W // 2, 2, C)
    return jnp.maximum(vr[:, :, :, 0], vr[:, :, :, 1])


def _convnet_kernel(x_ref, *refs):
    o_ref = refs[-1]
    x = x_ref[...]
    ci = 0
    for v in _CFG:
        if v == 'M':
            x = _pool_layer(x)
        else:
            x = _conv_layer(x, refs[2 * ci][...], refs[2 * ci + 1][...])
            ci += 1
    o_ref[...] = x.reshape(x.shape[0], x.shape[3])


def _convnet(xh, conv_ws, conv_bs):
    B = xh.shape[0]
    Bg = 8
    in_specs = [pl.BlockSpec((Bg, 32, 32, xh.shape[3]),
                             lambda i: (i, 0, 0, 0))]
    args = [xh]
    for w, b in zip(conv_ws, conv_bs):
        in_specs.append(pl.BlockSpec(w.shape, lambda i: (0, 0, 0, 0)))
        in_specs.append(pl.BlockSpec(b.shape, lambda i: (0, 0)))
        args.append(w)
        args.append(b)
    return pl.pallas_call(
        _convnet_kernel,
        out_shape=jax.ShapeDtypeStruct((B, 512), jnp.bfloat16),
        grid=(B // Bg,),
        in_specs=in_specs,
        out_specs=pl.BlockSpec((Bg, 512), lambda i: (i, 0)),
        compiler_params=pltpu.CompilerParams(
            dimension_semantics=("parallel",),
            vmem_limit_bytes=_VMEM_LIMIT,
        ),
    )(*args)


def _fc_kernel(x_ref, w_ref, b_ref, o_ref, acc_ref, *, relu):
    k = pl.program_id(1)

    @pl.when(k == 0)
    def _init():
        acc_ref[...] = jnp.zeros_like(acc_ref)

    acc_ref[...] += jnp.dot(x_ref[...], w_ref[...],
                            preferred_element_type=jnp.float32)

    @pl.when(k == pl.num_programs(1) - 1)
    def _fin():
        r = acc_ref[...] + b_ref[...]
        if relu:
            r = jnp.maximum(r, 0.0)
        o_ref[...] = r.astype(o_ref.dtype)


def _fc(x, w, b, *, relu, out_dtype, tk, tn):
    M, K = x.shape
    N = w.shape[1]
    b2 = b.astype(jnp.float32).reshape(1, N)
    grid = (N // tn, K // tk)
    return pl.pallas_call(
        functools.partial(_fc_kernel, relu=relu),
        out_shape=jax.ShapeDtypeStruct((M, N), out_dtype),
        grid=grid,
        in_specs=[
            pl.BlockSpec((M, tk), lambda j, k: (0, k)),
            pl.BlockSpec((tk, tn), lambda j, k: (k, j)),
            pl.BlockSpec((1, tn), lambda j, k: (0, j)),
        ],
        out_specs=pl.BlockSpec((M, tn), lambda j, k: (0, j)),
        scratch_shapes=[pltpu.VMEM((M, tn), jnp.float32)],
        compiler_params=pltpu.CompilerParams(
            dimension_semantics=("parallel", "arbitrary"),
            vmem_limit_bytes=_VMEM_LIMIT,
        ),
    )(x, w, b2)


def kernel(conv_w_0, conv_b_0, conv_w_1, conv_b_1, conv_w_2, conv_b_2,
           conv_w_3, conv_b_3, conv_w_4, conv_b_4, conv_w_5, conv_b_5,
           conv_w_6, conv_b_6, conv_w_7, conv_b_7, conv_w_8, conv_b_8,
           conv_w_9, conv_b_9, conv_w_10, conv_b_10, conv_w_11, conv_b_11,
           conv_w_12, conv_b_12, fc_w_0, fc_b_0, fc_w_1, fc_b_1,
           fc_w_2, fc_b_2, x):
    conv_ws = [conv_w_0, conv_w_1, conv_w_2, conv_w_3, conv_w_4, conv_w_5,
               conv_w_6, conv_w_7, conv_w_8, conv_w_9, conv_w_10, conv_w_11,
               conv_w_12]
    conv_bs = [conv_b_0, conv_b_1, conv_b_2, conv_b_3, conv_b_4, conv_b_5,
               conv_b_6, conv_b_7, conv_b_8, conv_b_9, conv_b_10, conv_b_11,
               conv_b_12]
    conv_ws = [w.astype(jnp.bfloat16) for w in conv_ws]
    conv_bs = [b.astype(jnp.float32).reshape(1, -1) for b in conv_bs]
    conv_ws[0] = jnp.pad(conv_ws[0], ((0, 0), (0, 0), (0, 5), (0, 0)))
    xh = jnp.transpose(x, (0, 2, 3, 1)).astype(jnp.bfloat16)
    xh = jnp.pad(xh, ((0, 0), (0, 0), (0, 0), (0, 5)))

    feat = _convnet(xh, conv_ws, conv_bs)

    xfc = jnp.repeat(feat, 49, axis=1)

    h = _fc(xfc, fc_w_0.astype(jnp.bfloat16), fc_b_0, relu=True,
            out_dtype=jnp.bfloat16, tk=3584, tn=2048)
    h = _fc(h, fc_w_1.astype(jnp.bfloat16), fc_b_1, relu=True,
            out_dtype=jnp.bfloat16, tk=2048, tn=2048)
    out = _fc(h, fc_w_2.astype(jnp.bfloat16), fc_b_2, relu=False,
              out_dtype=jnp.float32, tk=2048, tn=128)
    return out[:, :10]


# speedup vs baseline: 1.9773x; 1.4376x over previous
import functools

import jax
import jax.numpy as jnp
from jax.experimental import pallas as pl
from jax.experimental.pallas import tpu as pltpu

_CFG = [64, 64, 'M', 128, 128, 'M', 256, 256, 256, 'M',
        512, 512, 512, 'M', 512, 512, 512, 'M']

_VMEM_LIMIT = 48 * 1024 * 1024


def _conv_layer(x, w, b):
    Bg, H, W, Cin = x.shape
    Cout = w.shape[-1]
    Mp = Bg * (H + 2) * W
    xp = jnp.pad(x, ((0, 0), (1, 1), (1, 1), (0, 0)))
    xcat = jnp.concatenate([xp[:, :, kw:kw + W, :] for kw in range(3)],
                           axis=3).reshape(Mp, 3 * Cin)
    xcat = jnp.concatenate(
        [xcat, jnp.zeros((2 * W, 3 * Cin), jnp.bfloat16)], axis=0)
    acc = None
    for kh in range(3):
        d = jnp.dot(xcat[kh * W:kh * W + Mp, :], w[kh],
                    preferred_element_type=jnp.float32)
        acc = d if acc is None else acc + d
    out = jnp.maximum(acc + b, 0.0).astype(jnp.bfloat16)
    return out.reshape(Bg, H + 2, W, Cout)[:, :H]


def _pool_layer(x):
    Bg, H, W, C = x.shape
    xr = x.reshape(Bg, H // 2, 2, W, C)
    v = jnp.maximum(xr[:, :, 0], xr[:, :, 1])
    vr = v.reshape(Bg, H // 2, W // 2, 2, C)
    return jnp.maximum(vr[:, :, :, 0], vr[:, :, :, 1])


def _convnet_kernel(x_ref, *refs):
    o_ref = refs[-1]
    x = x_ref[...]
    ci = 0
    for v in _CFG:
        if v == 'M':
            x = _pool_layer(x)
        else:
            x = _conv_layer(x, refs[2 * ci][...], refs[2 * ci + 1][...])
            ci += 1
    o_ref[...] = x.reshape(x.shape[0], x.shape[3])


def _convnet(xh, conv_ws, conv_bs):
    B = xh.shape[0]
    Bg = 8
    in_specs = [pl.BlockSpec((Bg, 32, 32, xh.shape[3]),
                             lambda i: (i, 0, 0, 0))]
    args = [xh]
    for w, b in zip(conv_ws, conv_bs):
        in_specs.append(pl.BlockSpec(w.shape, lambda i, n=w.ndim: (0,) * n))
        in_specs.append(pl.BlockSpec(b.shape, lambda i: (0, 0)))
        args.append(w)
        args.append(b)
    return pl.pallas_call(
        _convnet_kernel,
        out_shape=jax.ShapeDtypeStruct((B, 512), jnp.bfloat16),
        grid=(B // Bg,),
        in_specs=in_specs,
        out_specs=pl.BlockSpec((Bg, 512), lambda i: (i, 0)),
        compiler_params=pltpu.CompilerParams(
            dimension_semantics=("parallel",),
            vmem_limit_bytes=_VMEM_LIMIT,
        ),
    )(*args)


def _fc_kernel(x_ref, w_ref, b_ref, o_ref, acc_ref, *, relu, rep):
    k = pl.program_id(1)

    @pl.when(k == 0)
    def _init():
        acc_ref[...] = jnp.zeros_like(acc_ref)

    xb = x_ref[0] if rep > 1 else x_ref[...]
    if rep > 1:
        kc = xb.shape[1]
        tk = kc * rep
        row = jax.lax.broadcasted_iota(jnp.int32, (kc, tk), 0)
        col = jax.lax.broadcasted_iota(jnp.int32, (kc, tk), 1)
        sel = (col // rep == row).astype(jnp.bfloat16)
        xb = jnp.dot(xb, sel,
                     preferred_element_type=jnp.float32).astype(jnp.bfloat16)
    acc_ref[...] += jnp.dot(xb, w_ref[...],
                            preferred_element_type=jnp.float32)

    @pl.when(k == pl.num_programs(1) - 1)
    def _fin():
        r = acc_ref[...] + b_ref[...]
        if relu:
            r = jnp.maximum(r, 0.0)
        o_ref[...] = r.astype(o_ref.dtype)


def _fc(x, w, b, *, relu, out_dtype, tk, tn, rep=1):
    M, K = x.shape
    N = w.shape[1]
    b2 = b.astype(jnp.float32).reshape(1, N)
    grid = (N // tn, K * rep // tk)
    if rep > 1:
        kc = tk // rep
        x = jnp.swapaxes(x.reshape(M, K // kc, kc), 0, 1)
        x_spec = pl.BlockSpec((1, M, kc), lambda j, k: (k, 0, 0))
    else:
        x_spec = pl.BlockSpec((M, tk), lambda j, k: (0, k))
    return pl.pallas_call(
        functools.partial(_fc_kernel, relu=relu, rep=rep),
        out_shape=jax.ShapeDtypeStruct((M, N), out_dtype),
        grid=grid,
        in_specs=[
            x_spec,
            pl.BlockSpec((tk, tn), lambda j, k: (k, j)),
            pl.BlockSpec((1, tn), lambda j, k: (0, j)),
        ],
        out_specs=pl.BlockSpec((M, tn), lambda j, k: (0, j)),
        scratch_shapes=[pltpu.VMEM((M, tn), jnp.float32)],
        compiler_params=pltpu.CompilerParams(
            dimension_semantics=("parallel", "arbitrary"),
            vmem_limit_bytes=_VMEM_LIMIT,
        ),
    )(x, w, b2)


def kernel(conv_w_0, conv_b_0, conv_w_1, conv_b_1, conv_w_2, conv_b_2,
           conv_w_3, conv_b_3, conv_w_4, conv_b_4, conv_w_5, conv_b_5,
           conv_w_6, conv_b_6, conv_w_7, conv_b_7, conv_w_8, conv_b_8,
           conv_w_9, conv_b_9, conv_w_10, conv_b_10, conv_w_11, conv_b_11,
           conv_w_12, conv_b_12, fc_w_0, fc_b_0, fc_w_1, fc_b_1,
           fc_w_2, fc_b_2, x):
    conv_ws = [conv_w_0, conv_w_1, conv_w_2, conv_w_3, conv_w_4, conv_w_5,
               conv_w_6, conv_w_7, conv_w_8, conv_w_9, conv_w_10, conv_w_11,
               conv_w_12]
    conv_bs = [conv_b_0, conv_b_1, conv_b_2, conv_b_3, conv_b_4, conv_b_5,
               conv_b_6, conv_b_7, conv_b_8, conv_b_9, conv_b_10, conv_b_11,
               conv_b_12]
    conv_ws = [w.astype(jnp.bfloat16) for w in conv_ws]
    conv_bs = [b.astype(jnp.float32).reshape(1, -1) for b in conv_bs]
    conv_ws[0] = jnp.pad(conv_ws[0], ((0, 0), (0, 0), (0, 5), (0, 0)))
    conv_ws = [jnp.swapaxes(w, 0, 1).reshape(3, -1, w.shape[3])
               for w in conv_ws]
    xh = jnp.transpose(x, (0, 2, 3, 1)).astype(jnp.bfloat16)
    xh = jnp.pad(xh, ((0, 0), (0, 0), (0, 0), (0, 5)))

    feat = _convnet(xh, conv_ws, conv_bs)

    h = _fc(feat, fc_w_0.astype(jnp.bfloat16), fc_b_0, relu=True,
            out_dtype=jnp.bfloat16, tk=3136, tn=2048, rep=49)
    h = _fc(h, fc_w_1.astype(jnp.bfloat16), fc_b_1, relu=True,
            out_dtype=jnp.bfloat16, tk=2048, tn=2048)
    out = _fc(h, fc_w_2.astype(jnp.bfloat16), fc_b_2, relu=False,
              out_dtype=jnp.float32, tk=2048, tn=128)
    return out[:, :10]
